# baseline (speedup 1.0000x reference)
import jax
import jax.numpy as jnp
from jax.experimental import pallas as pl
from jax.experimental.pallas import tpu as pltpu

_NUM_CLUSTERS = 4096
_K = 1024
_C = 256
_B = 32
_H = 100
_W = 100
_MAX_DIST = 50.0
_NUM_BINS = 50
_HID = 512


_BB = 2


def _score_sort_body(ego_ref, bins_ref, f00_ref, f01_ref, f10_ref, f11_ref,
                     du_ref, dv_ref, goals_ref, w1_ref, b1_ref,
                     w2_ref, b2_ref, selg_ref, sels_ref):
    pid = pl.program_id(0)
    goals = goals_ref[...]
    du = du_ref[...]
    dv = dv_ref[...]
    omdu = 1.0 - du
    omdv = 1.0 - dv
    jr = jax.lax.broadcasted_iota(jnp.int32, (_K, _K), 0)
    jc = jax.lax.broadcasted_iota(jnp.int32, (_K, _K), 1)
    tri = jc < jr
    iota_bins = jax.lax.broadcasted_iota(jnp.int32, (_K, _NUM_BINS), 1)

    for i in range(_BB):
        b = pid * _BB + i
        top = f00_ref[i] * omdu + f01_ref[i] * du
        bot = f10_ref[i] * omdu + f11_ref[i] * du
        feat_t = top * omdv + bot * dv
        h = jax.nn.relu(
            jax.lax.dot_general(feat_t, w1_ref[...],
                                (((0,), (0,)), ((), ())))
            + b1_ref[...])
        driv = (jax.lax.dot_general(h, w2_ref[...], (((1,), (0,)), ((), ())))
                + b2_ref[0, 0])
        ex = ego_ref[b, 0]
        ey = ego_ref[b, 1]
        dx = goals[:, 0:1] - ex
        dy = goals[:, 1:2] - ey
        dist = jnp.sqrt(dx * dx + dy * dy + 1e-8)
        bin_idx = jnp.clip((dist / _MAX_DIST * _NUM_BINS).astype(jnp.int32),
                           0, _NUM_BINS - 1)
        bsel = jnp.where(bin_idx == iota_bins, bins_ref[...], 0.0)
        dist_score = jnp.sum(bsel, axis=1, keepdims=True)

        s_col = dist_score + driv
        s_row = jnp.transpose(s_col)

        beats = (s_row > s_col) | ((s_row == s_col) & tri)
        rank_col = jnp.sum(jnp.where(beats, 1.0, 0.0), axis=1,
                           keepdims=True)
        rank_row = jnp.transpose(rank_col)

        p_mat = jnp.where(jr == rank_row.astype(jnp.int32), 1.0, 0.0)
        src = jnp.concatenate([goals, s_col], axis=1)
        out3 = jax.lax.dot_general(
            p_mat, src, (((1,), (0,)), ((), ())),
            precision=jax.lax.Precision.HIGHEST)
        selg_ref[i] = out3[:, 0:2]
        sels_ref[i] = jnp.transpose(out3[:, 2:3])


@jax.jit
def _score_sort(ego_state, f00, f01, f10, f11, du, dv, goals,
                distance_bin_scores, w1, b1, w2, b2):
    grid = (_B // _BB,)
    selg, sels = pl.pallas_call(
        _score_sort_body,
        grid=grid,
        in_specs=[
            pl.BlockSpec(memory_space=pltpu.SMEM),
            pl.BlockSpec((1, _NUM_BINS), lambda b: (0, 0)),
            pl.BlockSpec((_BB, _C, _K), lambda b: (b, 0, 0)),
            pl.BlockSpec((_BB, _C, _K), lambda b: (b, 0, 0)),
            pl.BlockSpec((_BB, _C, _K), lambda b: (b, 0, 0)),
            pl.BlockSpec((_BB, _C, _K), lambda b: (b, 0, 0)),
            pl.BlockSpec((1, _K), lambda b: (0, 0)),
            pl.BlockSpec((1, _K), lambda b: (0, 0)),
            pl.BlockSpec((_K, 2), lambda b: (0, 0)),
            pl.BlockSpec((_C, _HID), lambda b: (0, 0)),
            pl.BlockSpec((1, _HID), lambda b: (0, 0)),
            pl.BlockSpec((_HID, 1), lambda b: (0, 0)),
            pl.BlockSpec(memory_space=pltpu.SMEM),
        ],
        out_specs=[
            pl.BlockSpec((_BB, _K, 2), lambda b: (b, 0, 0)),
            pl.BlockSpec((_BB, 1, _K), lambda b: (b, 0, 0)),
        ],
        out_shape=[
            jax.ShapeDtypeStruct((_B, _K, 2), jnp.float32),
            jax.ShapeDtypeStruct((_B, 1, _K), jnp.float32),
        ],
        compiler_params=pltpu.CompilerParams(
            dimension_semantics=("arbitrary",)),
    )(ego_state, distance_bin_scores.reshape(1, _NUM_BINS),
      f00, f01, f10, f11, du.reshape(1, _K), dv.reshape(1, _K), goals,
      w1, b1.reshape(1, _HID), w2, b2.reshape(1, 1))
    return selg, sels.reshape(_B, _K)


def kernel(ego_state, bev_features, cluster_centers, distance_bin_scores,
           w1, b1, w2, b2):
    perm = jax.random.permutation(jax.random.key(42), _NUM_CLUSTERS)[:_K]
    goals = jnp.take(cluster_centers, perm, axis=0)
    u = jnp.clip((goals[:, 0] + _MAX_DIST) / (2.0 * _MAX_DIST) * (_W - 1),
                 0.0, _W - 1.0)
    v = jnp.clip((goals[:, 1] + _MAX_DIST) / (2.0 * _MAX_DIST) * (_H - 1),
                 0.0, _H - 1.0)
    u0 = jnp.floor(u).astype(jnp.int32)
    v0 = jnp.floor(v).astype(jnp.int32)
    u1 = jnp.clip(u0 + 1, 0, _W - 1)
    v1 = jnp.clip(v0 + 1, 0, _H - 1)
    u0c = jnp.clip(u0, 0, _W - 1)
    v0c = jnp.clip(v0, 0, _H - 1)
    du = u - u0c.astype(u.dtype)
    dv = v - v0c.astype(v.dtype)
    f00 = bev_features[:, :, v0c, u0c]
    f01 = bev_features[:, :, v0c, u1]
    f10 = bev_features[:, :, v1, u0c]
    f11 = bev_features[:, :, v1, u1]

    selected_goals, selected_scores = _score_sort(
        ego_state, f00, f01, f10, f11, du, dv, goals,
        distance_bin_scores, w1, b1, w2, b2)
    candidate_goals = jnp.broadcast_to(goals[None, :, :], (_B, _K, 2))
    return selected_goals, selected_scores, candidate_goals
